# Initial kernel scaffold; baseline (speedup 1.0000x reference)
import jax
import jax.numpy as jnp
from jax import lax
from jax.experimental import pallas as pl
from jax.experimental.pallas import tpu as pltpu
from jax.experimental.pallas import tpu_sc as plsc

D_DIM = 32
B_FLAT = 4096 * 200
NUM_CORES = 2
NUM_SUBCORES = 16
NUM_WORKERS = NUM_CORES * NUM_SUBCORES
B_PER_W = B_FLAT // NUM_WORKERS
CHUNK = 1024
N_CHUNKS = B_PER_W // CHUNK


def _emb_body(table_hbm, idx_hbm, out_hbm, idx_v, rows_v, sem):
    c = lax.axis_index("c")
    s = lax.axis_index("s")
    wid = s * NUM_CORES + c
    base = wid * B_PER_W

    def body(g, carry):
        off = base + g * CHUNK
        pltpu.sync_copy(idx_hbm.at[pl.ds(off, CHUNK)], idx_v)
        pltpu.async_copy(table_hbm.at[idx_v], rows_v, sem).wait()
        pltpu.sync_copy(rows_v, out_hbm.at[pl.ds(off, CHUNK)])
        return carry

    lax.fori_loop(0, N_CHUNKS, body, 0)


def kernel(weight, token_ids):
    idx = token_ids.reshape(-1).astype(jnp.int32)
    mesh = plsc.VectorSubcoreMesh(core_axis_name="c", subcore_axis_name="s")
    out = pl.kernel(
        _emb_body,
        out_type=jax.ShapeDtypeStruct((B_FLAT, D_DIM), jnp.float32),
        mesh=mesh,
        scratch_types=[
            pltpu.VMEM((CHUNK,), jnp.int32),
            pltpu.VMEM((CHUNK, D_DIM), jnp.float32),
            pltpu.SemaphoreType.DMA,
        ],
    )(weight, idx)
    return out.reshape(token_ids.shape + (D_DIM,))


# speedup vs baseline: 1.4581x; 1.4581x over previous
import jax
import jax.numpy as jnp
from jax import lax
from jax.experimental import pallas as pl
from jax.experimental.pallas import tpu as pltpu
from jax.experimental.pallas import tpu_sc as plsc

D_DIM = 32
B_FLAT = 4096 * 200
NUM_CORES = 2
NUM_SUBCORES = 16
NUM_WORKERS = NUM_CORES * NUM_SUBCORES
B_PER_W = B_FLAT // NUM_WORKERS
CHUNK = 1024
N_CHUNKS = B_PER_W // CHUNK


def _emb_body(table_hbm, idx_hbm, out_hbm, idx_v, rows_v, sem):
    c = lax.axis_index("c")
    s = lax.axis_index("s")
    wid = s * NUM_CORES + c
    base = wid * B_PER_W

    def body(g, carry):
        off = base + g * CHUNK
        pltpu.sync_copy(idx_hbm.at[pl.ds(off, CHUNK)], idx_v)
        pltpu.async_copy(table_hbm.at[idx_v], rows_v, sem).wait()
        pltpu.sync_copy(rows_v, out_hbm.at[pl.ds(off, CHUNK)])
        return carry

    lax.fori_loop(0, N_CHUNKS, body, 0)


def kernel(weight, token_ids):
    idx = token_ids.reshape(-1).astype(jnp.int32)
    mesh = plsc.VectorSubcoreMesh(core_axis_name="c", subcore_axis_name="s")
    out = pl.kernel(
        _emb_body,
        out_type=jax.ShapeDtypeStruct((B_FLAT, D_DIM), jnp.float32),
        mesh=mesh,
        scratch_types=[
            pltpu.VMEM((CHUNK,), jnp.int32),
            pltpu.VMEM((CHUNK, D_DIM), jnp.float32),
            pltpu.SemaphoreType.DMA,
        ],
        compiler_params=pltpu.CompilerParams(use_tc_tiling_on_sc=False),
    )(weight, idx)
    return out.reshape(token_ids.shape + (D_DIM,))


# speedup vs baseline: 1.4996x; 1.0285x over previous
import jax
import jax.numpy as jnp
from jax import lax
from jax.experimental import pallas as pl
from jax.experimental.pallas import tpu as pltpu
from jax.experimental.pallas import tpu_sc as plsc

D_DIM = 32
B_FLAT = 4096 * 200
NUM_CORES = 2
NUM_SUBCORES = 16
NUM_WORKERS = NUM_CORES * NUM_SUBCORES
B_PER_W = B_FLAT // NUM_WORKERS
CHUNK = 1024
N_CHUNKS = B_PER_W // CHUNK
NBUF = 3


def _emb_body(table_hbm, idx_hbm, out_hbm, idx_v, rows_v,
              gsem0, gsem1, gsem2, ssem0, ssem1, ssem2):
    c = lax.axis_index("c")
    s = lax.axis_index("s")
    wid = s * NUM_CORES + c
    base = wid * B_PER_W

    gsems = [gsem0, gsem1, gsem2]
    ssems = [ssem0, ssem1, ssem2]

    pltpu.sync_copy(idx_hbm.at[wid], idx_v)

    def gather_start(g, b):
        pltpu.async_copy(table_hbm.at[idx_v.at[g]], rows_v.at[b], gsems[b])

    def gather_wait(g, b):
        pltpu.make_async_copy(table_hbm.at[idx_v.at[g]], rows_v.at[b],
                              gsems[b]).wait()

    def out_slice(g):
        return out_hbm.at[pl.ds(base + g * CHUNK, CHUNK)]

    def store_start(g, b):
        pltpu.async_copy(rows_v.at[b], out_slice(g), ssems[b])

    def store_wait(g, b):
        pltpu.make_async_copy(rows_v.at[b], out_slice(g), ssems[b]).wait()

    for g in range(N_CHUNKS):
        b = g % NBUF
        if g >= NBUF:
            store_wait(g - NBUF, b)
        gather_start(g, b)
        if g >= 1:
            gb = (g - 1) % NBUF
            gather_wait(g - 1, gb)
            store_start(g - 1, gb)

    g_last = N_CHUNKS - 1
    gather_wait(g_last, g_last % NBUF)
    store_start(g_last, g_last % NBUF)
    for g in range(max(N_CHUNKS - NBUF, 0), N_CHUNKS):
        store_wait(g, g % NBUF)


def kernel(weight, token_ids):
    idx = token_ids.reshape(NUM_WORKERS, N_CHUNKS, CHUNK).astype(jnp.int32)
    mesh = plsc.VectorSubcoreMesh(core_axis_name="c", subcore_axis_name="s")
    out = pl.kernel(
        _emb_body,
        out_type=jax.ShapeDtypeStruct((B_FLAT, D_DIM), jnp.float32),
        mesh=mesh,
        scratch_types=[
            pltpu.VMEM((N_CHUNKS, CHUNK), jnp.int32),
            pltpu.VMEM((NBUF, CHUNK, D_DIM), jnp.float32),
            pltpu.SemaphoreType.DMA,
            pltpu.SemaphoreType.DMA,
            pltpu.SemaphoreType.DMA,
            pltpu.SemaphoreType.DMA,
            pltpu.SemaphoreType.DMA,
            pltpu.SemaphoreType.DMA,
        ],
        compiler_params=pltpu.CompilerParams(use_tc_tiling_on_sc=False),
    )(weight, idx)
    return out.reshape(token_ids.shape + (D_DIM,))


# speedup vs baseline: 1.4999x; 1.0002x over previous
import jax
import jax.numpy as jnp
from jax import lax
from jax.experimental import pallas as pl
from jax.experimental.pallas import tpu as pltpu
from jax.experimental.pallas import tpu_sc as plsc

D_DIM = 32
B_FLAT = 4096 * 200
NUM_CORES = 2
NUM_SUBCORES = 16
NUM_WORKERS = NUM_CORES * NUM_SUBCORES
B_PER_W = B_FLAT // NUM_WORKERS
CHUNK = 512
N_CHUNKS = B_PER_W // CHUNK
NBUF = 6
K_INFLIGHT = 4


def _emb_body(table_hbm, idx_hbm, out_hbm, idx_v, rows_v, gsem, ssem):
    c = lax.axis_index("c")
    s = lax.axis_index("s")
    wid = s * NUM_CORES + c
    base = wid * B_PER_W

    pltpu.sync_copy(idx_hbm.at[wid], idx_v)

    def gather_start(g, b):
        pltpu.async_copy(table_hbm.at[idx_v.at[g]], rows_v.at[b],
                         gsem.at[b])

    def gather_wait(g, b):
        pltpu.make_async_copy(table_hbm.at[idx_v.at[g]], rows_v.at[b],
                              gsem.at[b]).wait()

    def out_slice(g):
        return out_hbm.at[pl.ds(base + g * CHUNK, CHUNK)]

    def store_start(g, b):
        pltpu.async_copy(rows_v.at[b], out_slice(g), ssem.at[b])

    def store_wait(g, b):
        pltpu.make_async_copy(rows_v.at[b], out_slice(g), ssem.at[b]).wait()

    for g in range(N_CHUNKS):
        b = g % NBUF
        if g >= NBUF:
            store_wait(g - NBUF, b)
        gather_start(g, b)
        if g >= K_INFLIGHT:
            gd = g - K_INFLIGHT
            gather_wait(gd, gd % NBUF)
            store_start(gd, gd % NBUF)

    for g in range(max(N_CHUNKS - K_INFLIGHT, 0), N_CHUNKS):
        gather_wait(g, g % NBUF)
        store_start(g, g % NBUF)
    for g in range(max(N_CHUNKS - NBUF, 0), N_CHUNKS):
        store_wait(g, g % NBUF)


def kernel(weight, token_ids):
    idx = token_ids.reshape(NUM_WORKERS, N_CHUNKS, CHUNK).astype(jnp.int32)
    mesh = plsc.VectorSubcoreMesh(core_axis_name="c", subcore_axis_name="s")
    out = pl.kernel(
        _emb_body,
        out_type=jax.ShapeDtypeStruct((B_FLAT, D_DIM), jnp.float32),
        mesh=mesh,
        scratch_types=[
            pltpu.VMEM((N_CHUNKS, CHUNK), jnp.int32),
            pltpu.VMEM((NBUF, CHUNK, D_DIM), jnp.float32),
            pltpu.SemaphoreType.DMA((NBUF,)),
            pltpu.SemaphoreType.DMA((NBUF,)),
        ],
        compiler_params=pltpu.CompilerParams(use_tc_tiling_on_sc=False),
    )(weight, idx)
    return out.reshape(token_ids.shape + (D_DIM,))
